# baseline (speedup 1.0000x reference)
import functools

import jax
import jax.numpy as jnp
from jax import lax
from jax.experimental import pallas as pl
from jax.experimental.pallas import tpu as pltpu
import jax.experimental.pallas.tpu_sc as plsc

N = 10000
E = 320000
C = 128
NC = 2
NS = 16
NW = NC * NS
ECH = E // NC
EW = E // NW
B = 80
NB = EW // B
NSL = 5
B2 = 40
NB2 = EW // B2
NPAD = 10240
STR = NPAD // NS
T = 2000
NBLK = N // T

_mesh = plsc.VectorSubcoreMesh(core_axis_name="c", subcore_axis_name="s")


@functools.partial(
    pl.kernel,
    out_type=(
        jax.ShapeDtypeStruct((E, C), jnp.float32),
        jax.ShapeDtypeStruct((NW, C), jnp.float32),
        jax.ShapeDtypeStruct((NW, C), jnp.float32),
        jax.ShapeDtypeStruct((NC, NPAD), jnp.float32),
    ),
    mesh=_mesh,
    scratch_types=[
        pltpu.VMEM((NB, B), jnp.int32),
        pltpu.VMEM((NB, B), jnp.int32),
        pltpu.VMEM((B, C), jnp.float32),
        pltpu.VMEM((B, C), jnp.float32),
        pltpu.VMEM((B, C), jnp.float32),
        pltpu.VMEM((B, C), jnp.float32),
        pltpu.VMEM((B, C), jnp.float32),
        pltpu.VMEM((B, C), jnp.float32),
        pltpu.VMEM((B,), jnp.float32),
        pltpu.VMEM((STR,), jnp.float32),
        pltpu.VMEM((C,), jnp.float32),
        pltpu.VMEM((C,), jnp.float32),
        pltpu.VMEM_SHARED((NPAD,), jnp.float32),
        pltpu.SemaphoreType.DMA,
        pltpu.SemaphoreType.DMA,
        pltpu.SemaphoreType.DMA,
        pltpu.SemaphoreType.DMA,
        pltpu.SemaphoreType.DMA,
        pltpu.SemaphoreType.DMA,
    ],
)
def _sc_build_edges(xa_hbm, xb_hbm, src_hbm, dst_hbm,
                    h1_hbm, s1_hbm, s2_hbm, hist_hbm,
                    srcall, dstall,
                    bufa0, bufb0, out0, bufa1, bufb1, out1,
                    ones_v, zstr_v, acc1_v, acc2_v, hist_sh,
                    ga0, gb0, ga1, gb1, wb0, wb1):
    cid = lax.axis_index("c")
    sid = lax.axis_index("s")
    wid = cid * NS + sid
    cbase = cid * ECH + sid * EW

    for i in range(B // 16):
        ones_v[pl.ds(i * 16, 16)] = jnp.full((16,), 1.0, jnp.float32)

    def _zr(i, _):
        zstr_v[pl.ds(i * 16, 16)] = jnp.zeros((16,), jnp.float32)
        return 0

    lax.fori_loop(0, STR // 16, _zr, 0)
    pltpu.sync_copy(zstr_v, hist_sh.at[pl.ds(sid * STR, STR)])
    pltpu.sync_copy(src_hbm.at[wid], srcall)
    pltpu.sync_copy(dst_hbm.at[wid], dstall)
    plsc.subcore_barrier()

    def _compute(ba, bb, out, accs):
        def _row(i, accs):
            new = list(accs)
            for rr in range(2):
                r = 2 * i + rr
                for c in range(C // 16):
                    sl = pl.ds(c * 16, 16)
                    h = ba[r, sl] + bb[r, sl]
                    out[r, sl] = h
                    new[c] = new[c] + h
                    new[c + 8] = new[c + 8] + h * h
            return tuple(new)

        return lax.fori_loop(0, B // 2, _row, accs)

    def _fetch(b, ba, bb, sa, sb):
        pltpu.async_copy(xa_hbm.at[srcall.at[b]], ba, sa)
        pltpu.async_copy(xb_hbm.at[dstall.at[b]], bb, sb)

    def _gwait(ba, bb, sa, sb):
        pltpu.make_async_copy(xa_hbm.at[srcall.at[0]], ba, sa).wait()
        pltpu.make_async_copy(xb_hbm.at[dstall.at[0]], bb, sb).wait()

    _fetch(0, bufa0, bufb0, ga0, gb0)
    zv = jnp.zeros((16,), jnp.float32)

    def _pair(j, accs):
        _fetch(2 * j + 1, bufa1, bufb1, ga1, gb1)
        _gwait(bufa0, bufb0, ga0, gb0)
        pltpu.sync_copy(ones_v, hist_sh.at[dstall.at[2 * j]], add=True)

        @pl.when(j > 0)
        def _():
            pltpu.make_async_copy(out0, h1_hbm.at[pl.ds(0, B)], wb0).wait()

        accs = _compute(bufa0, bufb0, out0, accs)
        pltpu.async_copy(out0, h1_hbm.at[pl.ds(cbase + 2 * j * B, B)], wb0)
        _fetch(2 * j + 2, bufa0, bufb0, ga0, gb0)
        _gwait(bufa1, bufb1, ga1, gb1)
        pltpu.sync_copy(ones_v, hist_sh.at[dstall.at[2 * j + 1]], add=True)

        @pl.when(j > 0)
        def _():
            pltpu.make_async_copy(out1, h1_hbm.at[pl.ds(0, B)], wb1).wait()

        accs = _compute(bufa1, bufb1, out1, accs)
        pltpu.async_copy(out1, h1_hbm.at[pl.ds(cbase + (2 * j + 1) * B, B)],
                         wb1)
        return accs

    accs = lax.fori_loop(0, (NB - 1) // 2, _pair, (zv,) * 16)

    _gwait(bufa0, bufb0, ga0, gb0)
    pltpu.sync_copy(ones_v, hist_sh.at[dstall.at[NB - 1]], add=True)
    pltpu.make_async_copy(out0, h1_hbm.at[pl.ds(0, B)], wb0).wait()
    accs = _compute(bufa0, bufb0, out0, accs)
    pltpu.async_copy(out0, h1_hbm.at[pl.ds(cbase + (NB - 1) * B, B)], wb0)
    pltpu.make_async_copy(out0, h1_hbm.at[pl.ds(0, B)], wb0).wait()
    pltpu.make_async_copy(out1, h1_hbm.at[pl.ds(0, B)], wb1).wait()

    for c in range(C // 16):
        acc1_v[pl.ds(c * 16, 16)] = accs[c]
        acc2_v[pl.ds(c * 16, 16)] = accs[c + 8]
    pltpu.sync_copy(acc1_v, s1_hbm.at[wid])
    pltpu.sync_copy(acc2_v, s2_hbm.at[wid])
    plsc.subcore_barrier()
    pltpu.sync_copy(hist_sh.at[pl.ds(sid * STR, STR)],
                    hist_hbm.at[cid, pl.ds(sid * STR, STR)])


@functools.partial(
    pl.kernel,
    out_type=jax.ShapeDtypeStruct((NC, NPAD, C), jnp.float32),
    mesh=_mesh,
    scratch_types=(
        [pltpu.VMEM((B2,), jnp.int32) for _ in range(NSL)]
        + [pltpu.VMEM((B2,), jnp.int32) for _ in range(NSL)]
        + [pltpu.VMEM((B2, C), jnp.float32) for _ in range(NSL)]
        + [pltpu.VMEM((128, C), jnp.float32)]
        + [pltpu.VMEM_SHARED((NPAD, C), jnp.float32)]
        + [pltpu.SemaphoreType.DMA for _ in range(2 * NSL)]
    ),
)
def _sc_scatter_agg(p_hbm, src_hbm, dst_hbm, out_hbm, *refs):
    idxs = refs[0:NSL]
    idxd = refs[NSL:2 * NSL]
    rows = refs[2 * NSL:3 * NSL]
    zero_v = refs[3 * NSL]
    agg_sh = refs[3 * NSL + 1]
    gsem = refs[3 * NSL + 2:3 * NSL + 2 + NSL]
    ssem = refs[3 * NSL + 2 + NSL:3 * NSL + 2 + 2 * NSL]
    cid = lax.axis_index("c")
    sid = lax.axis_index("s")
    wid = cid * NS + sid

    def _zr(r, _):
        for c in range(C // 16):
            zero_v[r, pl.ds(c * 16, 16)] = jnp.zeros((16,), jnp.float32)
        return 0

    lax.fori_loop(0, 128, _zr, 0)
    for k in range(STR // 128):
        pltpu.sync_copy(zero_v, agg_sh.at[pl.ds(sid * STR + k * 128, 128)])
    plsc.subcore_barrier()

    def _fetch(b, s):
        pltpu.sync_copy(src_hbm.at[wid, b], idxs[s])
        pltpu.sync_copy(dst_hbm.at[wid, b], idxd[s])
        pltpu.async_copy(p_hbm.at[idxs[s]], rows[s], gsem[s])

    for s in range(NSL):
        _fetch(s, s)

    def _round(r, _):
        for s in range(NSL):
            pltpu.make_async_copy(p_hbm.at[idxs[s]], rows[s], gsem[s]).wait()
            pltpu.async_copy(rows[s], agg_sh.at[idxd[s]], ssem[s], add=True)
        for s in range(NSL):
            pltpu.make_async_copy(rows[s], agg_sh.at[idxd[s]], ssem[s]).wait()
            _fetch(NSL * r + NSL + s, s)
        return 0

    lax.fori_loop(0, NB2 // NSL - 1, _round, 0)

    for s in range(NSL):
        pltpu.make_async_copy(p_hbm.at[idxs[s]], rows[s], gsem[s]).wait()
        pltpu.async_copy(rows[s], agg_sh.at[idxd[s]], ssem[s], add=True)
    for s in range(NSL):
        pltpu.make_async_copy(rows[s], agg_sh.at[idxd[s]], ssem[s]).wait()

    plsc.subcore_barrier()
    for k in range(STR // 128):
        pltpu.sync_copy(agg_sh.at[pl.ds(sid * STR + k * 128, 128)],
                        out_hbm.at[cid, pl.ds(sid * STR + k * 128, 128)])


def _tc_xab(x, w1r):
    def body(x_ref, w_ref, o_ref):
        o_ref[...] = jnp.dot(x_ref[...], w_ref[...],
                             preferred_element_type=jnp.float32)

    return pl.pallas_call(
        body,
        out_shape=jax.ShapeDtypeStruct((N, 2 * C), jnp.float32),
    )(x, w1r)


def _tc_scale_head(hs, sdeg2d):
    def body(h_ref, s_ref, o_ref):
        o_ref[...] = h_ref[...] * s_ref[...]

    return pl.pallas_call(
        body,
        grid=(1,),
        in_specs=[
            pl.BlockSpec((N, C), lambda i: (0, 0)),
            pl.BlockSpec((N, 1), lambda i: (0, 0)),
        ],
        out_specs=pl.BlockSpec((N, C), lambda i: (0, 0)),
        out_shape=jax.ShapeDtypeStruct((N, C), jnp.float32),
    )(hs, sdeg2d)


def _tc_combine_agg1(aggp, h1, dis2d, inv2d, b1):
    def body(ap_ref, h_ref, d_ref, iv_ref, b_ref,
             agg_ref, q1_ref, q2_ref, q3_ref):
        a = (ap_ref[0, :N, :] + ap_ref[1, :N, :]) * d_ref[...]
        agg_ref[...] = a
        h = h_ref[...]
        iv = iv_ref[...]
        hi = h * iv
        g = hi + b_ref[...]
        q1_ref[...] = jnp.sum(hi - h, axis=0, keepdims=True)
        q2_ref[...] = jnp.sum(a, axis=0, keepdims=True)
        q3_ref[...] = jnp.sum(hi * hi - h * h + (2.0 * g + a) * a,
                              axis=0, keepdims=True)

    return pl.pallas_call(
        body,
        grid=(1,),
        in_specs=[
            pl.BlockSpec((NC, NPAD, C), lambda i: (0, 0, 0)),
            pl.BlockSpec((N, C), lambda i: (0, 0)),
            pl.BlockSpec((N, 1), lambda i: (0, 0)),
            pl.BlockSpec((N, 1), lambda i: (0, 0)),
            pl.BlockSpec((1, C), lambda i: (0, 0)),
        ],
        out_specs=[
            pl.BlockSpec((N, C), lambda i: (0, 0)),
            pl.BlockSpec((1, C), lambda i: (0, 0)),
            pl.BlockSpec((1, C), lambda i: (0, 0)),
            pl.BlockSpec((1, C), lambda i: (0, 0)),
        ],
        out_shape=[
            jax.ShapeDtypeStruct((N, C), jnp.float32),
            jax.ShapeDtypeStruct((1, C), jnp.float32),
            jax.ShapeDtypeStruct((1, C), jnp.float32),
            jax.ShapeDtypeStruct((1, C), jnp.float32),
        ],
    )(aggp, h1, dis2d, inv2d, b1)


def _tc_p2_head(h1, agg1, dis2d, inv2d, b1, s, t, w2):
    def body(h_ref, a_ref, d_ref, iv_ref, b_ref, s_ref, t_ref, w_ref, o_ref):
        o1 = h_ref[...] * iv_ref[...] + b_ref[...] + a_ref[...]
        z = jnp.maximum(o1 * s_ref[...] + t_ref[...], 0.0)
        o_ref[...] = jnp.dot(z * d_ref[...], w_ref[...],
                             preferred_element_type=jnp.float32)

    return pl.pallas_call(
        body,
        grid=(1,),
        in_specs=[
            pl.BlockSpec((N, C), lambda i: (0, 0)),
            pl.BlockSpec((N, C), lambda i: (0, 0)),
            pl.BlockSpec((N, 1), lambda i: (0, 0)),
            pl.BlockSpec((N, 1), lambda i: (0, 0)),
            pl.BlockSpec((1, C), lambda i: (0, 0)),
            pl.BlockSpec((1, C), lambda i: (0, 0)),
            pl.BlockSpec((1, C), lambda i: (0, 0)),
            pl.BlockSpec((C, C), lambda i: (0, 0)),
        ],
        out_specs=pl.BlockSpec((N, C), lambda i: (0, 0)),
        out_shape=jax.ShapeDtypeStruct((N, C), jnp.float32),
    )(h1, agg1, dis2d, inv2d, b1, s, t, w2)


def _tc_head(h1h, inv2d, agg1, agg2p, dis2d, s, t, b1, b2, w2, lnw, lnb, wlr):
    def body(h_ref, i_ref, a1_ref, a2_ref, d_ref, s_ref, t_ref, b1_ref,
             b2_ref, w2_ref, lw_ref, lb_ref, wl_ref, o_ref):
        o1 = h_ref[...] * i_ref[...] + b1_ref[...] + a1_ref[...]
        z = jnp.maximum(o1 * s_ref[...] + t_ref[...], 0.0)
        h2s = jnp.dot(z * i_ref[...], w2_ref[...],
                      preferred_element_type=jnp.float32)
        a2 = (a2_ref[0] + a2_ref[1]) * d_ref[...]
        o2 = h2s + b2_ref[...] + a2
        mu = jnp.mean(o2, axis=-1, keepdims=True)
        var = jnp.mean(o2 * o2, axis=-1, keepdims=True) - mu * mu
        zz = (o2 - mu) * lax.rsqrt(var + 1e-5) * lw_ref[...] + lb_ref[...]
        zz = jnp.maximum(zz, 0.0)
        o_ref[...] = jnp.sum(zz * wl_ref[...], axis=-1, keepdims=True)

    return pl.pallas_call(
        body,
        grid=(NBLK,),
        in_specs=[
            pl.BlockSpec((T, C), lambda i: (i, 0)),
            pl.BlockSpec((T, 1), lambda i: (i, 0)),
            pl.BlockSpec((T, C), lambda i: (i, 0)),
            pl.BlockSpec((NC, T, C), lambda i: (0, i, 0)),
            pl.BlockSpec((T, 1), lambda i: (i, 0)),
            pl.BlockSpec((1, C), lambda i: (0, 0)),
            pl.BlockSpec((1, C), lambda i: (0, 0)),
            pl.BlockSpec((1, C), lambda i: (0, 0)),
            pl.BlockSpec((1, C), lambda i: (0, 0)),
            pl.BlockSpec((C, C), lambda i: (0, 0)),
            pl.BlockSpec((1, C), lambda i: (0, 0)),
            pl.BlockSpec((1, C), lambda i: (0, 0)),
            pl.BlockSpec((1, C), lambda i: (0, 0)),
        ],
        out_specs=pl.BlockSpec((T, 1), lambda i: (i, 0)),
        out_shape=jax.ShapeDtypeStruct((N, 1), jnp.float32),
    )(h1h, inv2d, agg1, agg2p, dis2d, s, t, b1, b2, w2, lnw, lnb, wlr)


def _tc_tail(h1, sf, tf, b2, w2, w2m, red, b2m, lnw, lnb, wlr):
    def body(h_ref, s_ref, t_ref, b2_ref, w2_ref, wm_ref, rd_ref, bm_ref,
             lw_ref, lb_ref, wl_ref, o_ref):
        z = jnp.maximum(h_ref[...] * s_ref[...] + t_ref[...], 0.0)
        o2 = jnp.dot(z, w2_ref[...], preferred_element_type=jnp.float32)
        o2 = o2 + b2_ref[...]
        mu = jnp.dot(z, wm_ref[...],
                     preferred_element_type=jnp.float32) + bm_ref[...]
        msq = jnp.dot(o2 * o2, rd_ref[...], preferred_element_type=jnp.float32)
        var = msq - mu * mu
        zz = (o2 - mu) * lax.rsqrt(var + 1e-5) * lw_ref[...] + lb_ref[...]
        zz = jnp.maximum(zz, 0.0)
        o_ref[...] = jnp.dot(zz, wl_ref[...], preferred_element_type=jnp.float32)

    return pl.pallas_call(
        body,
        grid=((E - N) // T,),
        in_specs=[
            pl.BlockSpec((T, C), lambda i: (i + NBLK, 0)),
            pl.BlockSpec((1, C), lambda i: (0, 0)),
            pl.BlockSpec((1, C), lambda i: (0, 0)),
            pl.BlockSpec((1, C), lambda i: (0, 0)),
            pl.BlockSpec((C, C), lambda i: (0, 0)),
            pl.BlockSpec((C, 1), lambda i: (0, 0)),
            pl.BlockSpec((C, 1), lambda i: (0, 0)),
            pl.BlockSpec((1, 1), lambda i: (0, 0)),
            pl.BlockSpec((1, C), lambda i: (0, 0)),
            pl.BlockSpec((1, C), lambda i: (0, 0)),
            pl.BlockSpec((C, 1), lambda i: (0, 0)),
        ],
        out_specs=pl.BlockSpec((T, 1), lambda i: (i, 0)),
        out_shape=jax.ShapeDtypeStruct((E - N, 1), jnp.float32),
    )(h1, sf, tf, b2, w2, w2m, red, b2m, lnw, lnb, wlr)


def kernel(x, edge_index, W1, b1, bn_w, bn_b, W2, b2, ln_w, ln_b, Wl, bl):
    f32 = jnp.float32
    src = edge_index[0]
    dst = edge_index[1]

    w1r = jnp.concatenate([W1[:C], W1[C:]], axis=1)
    xab = _tc_xab(x, w1r)
    xa = xab[:, :C]
    xb = xab[:, C:]

    src3 = src.reshape(NW, NB, B)
    dst3 = dst.reshape(NW, NB, B)
    srcs = src.reshape(NW, NB2, B2)
    dsts = dst.reshape(NW, NB2, B2)
    h1, s1p, s2p, hist = _sc_build_edges(xa, xb, src3, dst3)
    u1 = jnp.sum(s1p, axis=0, keepdims=True)
    u2 = jnp.sum(s2p, axis=0, keepdims=True)
    h1_head = jax.lax.slice(h1, (0, 0), (N, C))

    deg = hist[0, :N] + hist[1, :N] + 1.0
    dis = lax.rsqrt(deg)
    inv = 1.0 / deg

    dis2d = dis[:, None]
    inv2d = inv[:, None]
    p1 = _tc_scale_head(h1_head, dis2d)
    agg1p = _sc_scatter_agg(p1, srcs, dsts)
    b1r = b1[None, :]
    agg1, q1, q2, q3 = _tc_combine_agg1(agg1p, h1_head, dis2d, inv2d, b1r)

    s1 = u1 + q1
    mean = (s1 + q2) / E + b1r
    ex2 = (u2 + q3 + 2.0 * b1r * s1) / E + b1r * b1r
    var = ex2 - mean * mean
    s = bn_w[None, :] * lax.rsqrt(var + 1e-5)
    t = bn_b[None, :] - mean * s

    p2 = _tc_p2_head(h1_head, agg1, dis2d, inv2d, b1r, s, t, W2)
    agg2p = _sc_scatter_agg(p2, srcs, dsts)

    b2r = b2[None, :]
    red = jnp.full((C, 1), 1.0 / C, f32)
    w2m = jnp.dot(W2, red)
    b2m = jnp.mean(b2).reshape(1, 1)
    tf = t + b1r * s
    y_tail = _tc_tail(h1, s, tf, b2r, W2, w2m, red, b2m,
                      ln_w[None, :], ln_b[None, :], Wl)
    y_head = _tc_head(h1_head, inv2d, agg1, agg2p, dis2d, s, t,
                      b1r, b2r, W2, ln_w[None, :], ln_b[None, :],
                      Wl.reshape(1, C))
    y = jnp.concatenate([y_head[:, 0], y_tail[:, 0]])
    return y + bl[0]


# speedup vs baseline: 1.2293x; 1.2293x over previous
import functools

import jax
import jax.numpy as jnp
from jax import lax
from jax.experimental import pallas as pl
from jax.experimental.pallas import tpu as pltpu
import jax.experimental.pallas.tpu_sc as plsc

N = 10000
E = 320000
C = 128
NC = 2
NS = 16
NW = NC * NS
ECH = E // NC
EW = E // NW
B = 80
NB = EW // B
NPAD = 10240
STR = NPAD // NS
T = 2000
NBLK = N // T

_mesh = plsc.VectorSubcoreMesh(core_axis_name="c", subcore_axis_name="s")


@functools.partial(
    pl.kernel,
    out_type=(
        jax.ShapeDtypeStruct((E, C), jnp.float32),
        jax.ShapeDtypeStruct((NW, C), jnp.float32),
        jax.ShapeDtypeStruct((NW, C), jnp.float32),
        jax.ShapeDtypeStruct((NC, NPAD), jnp.float32),
    ),
    mesh=_mesh,
    scratch_types=[
        pltpu.VMEM((NB, B), jnp.int32),
        pltpu.VMEM((NB, B), jnp.int32),
        pltpu.VMEM((B, C), jnp.float32),
        pltpu.VMEM((B, C), jnp.float32),
        pltpu.VMEM((B, C), jnp.float32),
        pltpu.VMEM((B, C), jnp.float32),
        pltpu.VMEM((B, C), jnp.float32),
        pltpu.VMEM((B, C), jnp.float32),
        pltpu.VMEM((B,), jnp.float32),
        pltpu.VMEM((STR,), jnp.float32),
        pltpu.VMEM((C,), jnp.float32),
        pltpu.VMEM((C,), jnp.float32),
        pltpu.VMEM_SHARED((NPAD,), jnp.float32),
        pltpu.SemaphoreType.DMA,
        pltpu.SemaphoreType.DMA,
        pltpu.SemaphoreType.DMA,
        pltpu.SemaphoreType.DMA,
        pltpu.SemaphoreType.DMA,
        pltpu.SemaphoreType.DMA,
    ],
)
def _sc_build_edges(xa_hbm, xb_hbm, src_hbm, dst_hbm,
                    h1_hbm, s1_hbm, s2_hbm, hist_hbm,
                    srcall, dstall,
                    bufa0, bufb0, out0, bufa1, bufb1, out1,
                    ones_v, zstr_v, acc1_v, acc2_v, hist_sh,
                    ga0, gb0, ga1, gb1, wb0, wb1):
    cid = lax.axis_index("c")
    sid = lax.axis_index("s")
    wid = cid * NS + sid
    cbase = cid * ECH + sid * EW

    for i in range(B // 16):
        ones_v[pl.ds(i * 16, 16)] = jnp.full((16,), 1.0, jnp.float32)

    def _zr(i, _):
        zstr_v[pl.ds(i * 16, 16)] = jnp.zeros((16,), jnp.float32)
        return 0

    lax.fori_loop(0, STR // 16, _zr, 0)
    pltpu.sync_copy(zstr_v, hist_sh.at[pl.ds(sid * STR, STR)])
    pltpu.sync_copy(src_hbm.at[wid], srcall)
    pltpu.sync_copy(dst_hbm.at[wid], dstall)
    plsc.subcore_barrier()

    def _compute(ba, bb, out, accs):
        def _row(i, accs):
            new = list(accs)
            for rr in range(2):
                r = 2 * i + rr
                for c in range(C // 16):
                    sl = pl.ds(c * 16, 16)
                    h = ba[r, sl] + bb[r, sl]
                    out[r, sl] = h
                    new[c] = new[c] + h
                    new[c + 8] = new[c + 8] + h * h
            return tuple(new)

        return lax.fori_loop(0, B // 2, _row, accs)

    def _fetch(b, ba, bb, sa, sb):
        pltpu.async_copy(xa_hbm.at[srcall.at[b]], ba, sa)
        pltpu.async_copy(xb_hbm.at[dstall.at[b]], bb, sb)

    def _gwait(ba, bb, sa, sb):
        pltpu.make_async_copy(xa_hbm.at[srcall.at[0]], ba, sa).wait()
        pltpu.make_async_copy(xb_hbm.at[dstall.at[0]], bb, sb).wait()

    _fetch(0, bufa0, bufb0, ga0, gb0)
    zv = jnp.zeros((16,), jnp.float32)

    def _pair(j, accs):
        _fetch(2 * j + 1, bufa1, bufb1, ga1, gb1)
        _gwait(bufa0, bufb0, ga0, gb0)
        pltpu.sync_copy(ones_v, hist_sh.at[dstall.at[2 * j]], add=True)

        @pl.when(j > 0)
        def _():
            pltpu.make_async_copy(out0, h1_hbm.at[pl.ds(0, B)], wb0).wait()

        accs = _compute(bufa0, bufb0, out0, accs)
        pltpu.async_copy(out0, h1_hbm.at[pl.ds(cbase + 2 * j * B, B)], wb0)
        _fetch(2 * j + 2, bufa0, bufb0, ga0, gb0)
        _gwait(bufa1, bufb1, ga1, gb1)
        pltpu.sync_copy(ones_v, hist_sh.at[dstall.at[2 * j + 1]], add=True)

        @pl.when(j > 0)
        def _():
            pltpu.make_async_copy(out1, h1_hbm.at[pl.ds(0, B)], wb1).wait()

        accs = _compute(bufa1, bufb1, out1, accs)
        pltpu.async_copy(out1, h1_hbm.at[pl.ds(cbase + (2 * j + 1) * B, B)],
                         wb1)
        return accs

    accs = lax.fori_loop(0, (NB - 1) // 2, _pair, (zv,) * 16)

    _gwait(bufa0, bufb0, ga0, gb0)
    pltpu.sync_copy(ones_v, hist_sh.at[dstall.at[NB - 1]], add=True)
    pltpu.make_async_copy(out0, h1_hbm.at[pl.ds(0, B)], wb0).wait()
    accs = _compute(bufa0, bufb0, out0, accs)
    pltpu.async_copy(out0, h1_hbm.at[pl.ds(cbase + (NB - 1) * B, B)], wb0)
    pltpu.make_async_copy(out0, h1_hbm.at[pl.ds(0, B)], wb0).wait()
    pltpu.make_async_copy(out1, h1_hbm.at[pl.ds(0, B)], wb1).wait()

    for c in range(C // 16):
        acc1_v[pl.ds(c * 16, 16)] = accs[c]
        acc2_v[pl.ds(c * 16, 16)] = accs[c + 8]
    pltpu.sync_copy(acc1_v, s1_hbm.at[wid])
    pltpu.sync_copy(acc2_v, s2_hbm.at[wid])
    plsc.subcore_barrier()
    pltpu.sync_copy(hist_sh.at[pl.ds(sid * STR, STR)],
                    hist_hbm.at[cid, pl.ds(sid * STR, STR)])


@functools.partial(
    pl.kernel,
    out_type=jax.ShapeDtypeStruct((NC, NPAD, C), jnp.float32),
    mesh=_mesh,
    scratch_types=[
        pltpu.VMEM((B,), jnp.int32),
        pltpu.VMEM((B,), jnp.int32),
        pltpu.VMEM((B, C), jnp.float32),
        pltpu.VMEM((B, C), jnp.float32),
        pltpu.VMEM((128, C), jnp.float32),
        pltpu.VMEM((B,), jnp.int32),
        pltpu.VMEM((B,), jnp.int32),
        pltpu.VMEM_SHARED((NPAD, C), jnp.float32),
        pltpu.SemaphoreType.DMA,
        pltpu.SemaphoreType.DMA,
        pltpu.SemaphoreType.DMA,
        pltpu.SemaphoreType.DMA,
    ],
)
def _sc_scatter_agg(p_hbm, src_hbm, dst_hbm, out_hbm,
                    idxs0, idxs1, rows0, rows1, zero_v,
                    idxd0, idxd1, agg_sh, g0, g1, sc0, sc1):
    cid = lax.axis_index("c")
    sid = lax.axis_index("s")
    wid = cid * NS + sid
    cbase = cid * ECH + sid * EW

    def _zr(r, _):
        for c in range(C // 16):
            zero_v[r, pl.ds(c * 16, 16)] = jnp.zeros((16,), jnp.float32)
        return 0

    lax.fori_loop(0, 128, _zr, 0)
    for k in range(STR // 128):
        pltpu.sync_copy(zero_v, agg_sh.at[pl.ds(sid * STR + k * 128, 128)])
    plsc.subcore_barrier()

    def _fetch(b, idxs, idxd, rows, sem):
        pltpu.sync_copy(src_hbm.at[wid, b], idxs)
        pltpu.sync_copy(dst_hbm.at[wid, b], idxd)
        pltpu.async_copy(p_hbm.at[idxs], rows, sem)

    def _gwait(idxs, rows, sem):
        pltpu.make_async_copy(p_hbm.at[idxs], rows, sem).wait()

    def _swait(idxd, rows, sem):
        pltpu.make_async_copy(rows, agg_sh.at[idxd], sem).wait()

    _fetch(0, idxs0, idxd0, rows0, g0)

    def _pair(j, _):
        @pl.when(j > 0)
        def _():
            _swait(idxd1, rows1, sc1)

        _fetch(2 * j + 1, idxs1, idxd1, rows1, g1)
        _gwait(idxs0, rows0, g0)
        pltpu.async_copy(rows0, agg_sh.at[idxd0], sc0, add=True)
        _swait(idxd0, rows0, sc0)
        _fetch(2 * j + 2, idxs0, idxd0, rows0, g0)
        _gwait(idxs1, rows1, g1)
        pltpu.async_copy(rows1, agg_sh.at[idxd1], sc1, add=True)
        return 0

    lax.fori_loop(0, (NB - 1) // 2, _pair, 0)

    _gwait(idxs0, rows0, g0)
    pltpu.async_copy(rows0, agg_sh.at[idxd0], sc0, add=True)
    _swait(idxd0, rows0, sc0)
    _swait(idxd1, rows1, sc1)

    plsc.subcore_barrier()
    for k in range(STR // 128):
        pltpu.sync_copy(agg_sh.at[pl.ds(sid * STR + k * 128, 128)],
                        out_hbm.at[cid, pl.ds(sid * STR + k * 128, 128)])


def _tc_xab(x, w1r):
    def body(x_ref, w_ref, o_ref):
        o_ref[...] = jnp.dot(x_ref[...], w_ref[...],
                             preferred_element_type=jnp.float32)

    return pl.pallas_call(
        body,
        out_shape=jax.ShapeDtypeStruct((N, 2 * C), jnp.float32),
    )(x, w1r)


def _tc_scale_head(hs, sdeg2d):
    def body(h_ref, s_ref, o_ref):
        o_ref[...] = h_ref[...] * s_ref[...]

    return pl.pallas_call(
        body,
        grid=(1,),
        in_specs=[
            pl.BlockSpec((N, C), lambda i: (0, 0)),
            pl.BlockSpec((N, 1), lambda i: (0, 0)),
        ],
        out_specs=pl.BlockSpec((N, C), lambda i: (0, 0)),
        out_shape=jax.ShapeDtypeStruct((N, C), jnp.float32),
    )(hs, sdeg2d)


def _tc_combine_agg1(aggp, h1, dis2d, inv2d, b1):
    def body(ap_ref, h_ref, d_ref, iv_ref, b_ref,
             agg_ref, q1_ref, q2_ref, q3_ref):
        a = (ap_ref[0, :N, :] + ap_ref[1, :N, :]) * d_ref[...]
        agg_ref[...] = a
        h = h_ref[...]
        iv = iv_ref[...]
        hi = h * iv
        g = hi + b_ref[...]
        q1_ref[...] = jnp.sum(hi - h, axis=0, keepdims=True)
        q2_ref[...] = jnp.sum(a, axis=0, keepdims=True)
        q3_ref[...] = jnp.sum(hi * hi - h * h + (2.0 * g + a) * a,
                              axis=0, keepdims=True)

    return pl.pallas_call(
        body,
        grid=(1,),
        in_specs=[
            pl.BlockSpec((NC, NPAD, C), lambda i: (0, 0, 0)),
            pl.BlockSpec((N, C), lambda i: (0, 0)),
            pl.BlockSpec((N, 1), lambda i: (0, 0)),
            pl.BlockSpec((N, 1), lambda i: (0, 0)),
            pl.BlockSpec((1, C), lambda i: (0, 0)),
        ],
        out_specs=[
            pl.BlockSpec((N, C), lambda i: (0, 0)),
            pl.BlockSpec((1, C), lambda i: (0, 0)),
            pl.BlockSpec((1, C), lambda i: (0, 0)),
            pl.BlockSpec((1, C), lambda i: (0, 0)),
        ],
        out_shape=[
            jax.ShapeDtypeStruct((N, C), jnp.float32),
            jax.ShapeDtypeStruct((1, C), jnp.float32),
            jax.ShapeDtypeStruct((1, C), jnp.float32),
            jax.ShapeDtypeStruct((1, C), jnp.float32),
        ],
    )(aggp, h1, dis2d, inv2d, b1)


def _tc_p2_head(h1, agg1, dis2d, inv2d, b1, s, t, w2):
    def body(h_ref, a_ref, d_ref, iv_ref, b_ref, s_ref, t_ref, w_ref, o_ref):
        o1 = h_ref[...] * iv_ref[...] + b_ref[...] + a_ref[...]
        z = jnp.maximum(o1 * s_ref[...] + t_ref[...], 0.0)
        o_ref[...] = jnp.dot(z * d_ref[...], w_ref[...],
                             preferred_element_type=jnp.float32)

    return pl.pallas_call(
        body,
        grid=(1,),
        in_specs=[
            pl.BlockSpec((N, C), lambda i: (0, 0)),
            pl.BlockSpec((N, C), lambda i: (0, 0)),
            pl.BlockSpec((N, 1), lambda i: (0, 0)),
            pl.BlockSpec((N, 1), lambda i: (0, 0)),
            pl.BlockSpec((1, C), lambda i: (0, 0)),
            pl.BlockSpec((1, C), lambda i: (0, 0)),
            pl.BlockSpec((1, C), lambda i: (0, 0)),
            pl.BlockSpec((C, C), lambda i: (0, 0)),
        ],
        out_specs=pl.BlockSpec((N, C), lambda i: (0, 0)),
        out_shape=jax.ShapeDtypeStruct((N, C), jnp.float32),
    )(h1, agg1, dis2d, inv2d, b1, s, t, w2)


def _tc_head(h1h, inv2d, agg1, agg2p, dis2d, s, t, b1, b2, w2, lnw, lnb, wlr):
    def body(h_ref, i_ref, a1_ref, a2_ref, d_ref, s_ref, t_ref, b1_ref,
             b2_ref, w2_ref, lw_ref, lb_ref, wl_ref, o_ref):
        o1 = h_ref[...] * i_ref[...] + b1_ref[...] + a1_ref[...]
        z = jnp.maximum(o1 * s_ref[...] + t_ref[...], 0.0)
        h2s = jnp.dot(z * i_ref[...], w2_ref[...],
                      preferred_element_type=jnp.float32)
        a2 = (a2_ref[0] + a2_ref[1]) * d_ref[...]
        o2 = h2s + b2_ref[...] + a2
        mu = jnp.mean(o2, axis=-1, keepdims=True)
        var = jnp.mean(o2 * o2, axis=-1, keepdims=True) - mu * mu
        zz = (o2 - mu) * lax.rsqrt(var + 1e-5) * lw_ref[...] + lb_ref[...]
        zz = jnp.maximum(zz, 0.0)
        o_ref[...] = jnp.sum(zz * wl_ref[...], axis=-1, keepdims=True)

    return pl.pallas_call(
        body,
        grid=(NBLK,),
        in_specs=[
            pl.BlockSpec((T, C), lambda i: (i, 0)),
            pl.BlockSpec((T, 1), lambda i: (i, 0)),
            pl.BlockSpec((T, C), lambda i: (i, 0)),
            pl.BlockSpec((NC, T, C), lambda i: (0, i, 0)),
            pl.BlockSpec((T, 1), lambda i: (i, 0)),
            pl.BlockSpec((1, C), lambda i: (0, 0)),
            pl.BlockSpec((1, C), lambda i: (0, 0)),
            pl.BlockSpec((1, C), lambda i: (0, 0)),
            pl.BlockSpec((1, C), lambda i: (0, 0)),
            pl.BlockSpec((C, C), lambda i: (0, 0)),
            pl.BlockSpec((1, C), lambda i: (0, 0)),
            pl.BlockSpec((1, C), lambda i: (0, 0)),
            pl.BlockSpec((1, C), lambda i: (0, 0)),
        ],
        out_specs=pl.BlockSpec((T, 1), lambda i: (i, 0)),
        out_shape=jax.ShapeDtypeStruct((N, 1), jnp.float32),
    )(h1h, inv2d, agg1, agg2p, dis2d, s, t, b1, b2, w2, lnw, lnb, wlr)


def _tc_tail(h1, sf, tf, b2, w2, w2m, red, b2m, lnw, lnb, wlr):
    def body(h_ref, s_ref, t_ref, b2_ref, w2_ref, wm_ref, rd_ref, bm_ref,
             lw_ref, lb_ref, wl_ref, o_ref):
        z = jnp.maximum(h_ref[...] * s_ref[...] + t_ref[...], 0.0)
        o2 = jnp.dot(z, w2_ref[...], preferred_element_type=jnp.float32)
        o2 = o2 + b2_ref[...]
        mu = jnp.dot(z, wm_ref[...],
                     preferred_element_type=jnp.float32) + bm_ref[...]
        msq = jnp.dot(o2 * o2, rd_ref[...], preferred_element_type=jnp.float32)
        var = msq - mu * mu
        zz = (o2 - mu) * lax.rsqrt(var + 1e-5) * lw_ref[...] + lb_ref[...]
        zz = jnp.maximum(zz, 0.0)
        o_ref[...] = jnp.dot(zz, wl_ref[...], preferred_element_type=jnp.float32)

    return pl.pallas_call(
        body,
        grid=((E - N) // T,),
        in_specs=[
            pl.BlockSpec((T, C), lambda i: (i + NBLK, 0)),
            pl.BlockSpec((1, C), lambda i: (0, 0)),
            pl.BlockSpec((1, C), lambda i: (0, 0)),
            pl.BlockSpec((1, C), lambda i: (0, 0)),
            pl.BlockSpec((C, C), lambda i: (0, 0)),
            pl.BlockSpec((C, 1), lambda i: (0, 0)),
            pl.BlockSpec((C, 1), lambda i: (0, 0)),
            pl.BlockSpec((1, 1), lambda i: (0, 0)),
            pl.BlockSpec((1, C), lambda i: (0, 0)),
            pl.BlockSpec((1, C), lambda i: (0, 0)),
            pl.BlockSpec((C, 1), lambda i: (0, 0)),
        ],
        out_specs=pl.BlockSpec((T, 1), lambda i: (i, 0)),
        out_shape=jax.ShapeDtypeStruct((E - N, 1), jnp.float32),
    )(h1, sf, tf, b2, w2, w2m, red, b2m, lnw, lnb, wlr)


def kernel(x, edge_index, W1, b1, bn_w, bn_b, W2, b2, ln_w, ln_b, Wl, bl):
    f32 = jnp.float32
    src = edge_index[0]
    dst = edge_index[1]

    w1r = jnp.concatenate([W1[:C], W1[C:]], axis=1)
    xab = _tc_xab(x, w1r)
    xa = xab[:, :C]
    xb = xab[:, C:]

    src3 = src.reshape(NW, NB, B)
    dst3 = dst.reshape(NW, NB, B)
    h1, s1p, s2p, hist = _sc_build_edges(xa, xb, src3, dst3)
    u1 = jnp.sum(s1p, axis=0, keepdims=True)
    u2 = jnp.sum(s2p, axis=0, keepdims=True)
    h1_head = jax.lax.slice(h1, (0, 0), (N, C))

    deg = hist[0, :N] + hist[1, :N] + 1.0
    dis = lax.rsqrt(deg)
    inv = 1.0 / deg

    dis2d = dis[:, None]
    inv2d = inv[:, None]
    p1 = _tc_scale_head(h1_head, dis2d)
    agg1p = _sc_scatter_agg(p1, src3, dst3)
    b1r = b1[None, :]
    agg1, q1, q2, q3 = _tc_combine_agg1(agg1p, h1_head, dis2d, inv2d, b1r)

    s1 = u1 + q1
    mean = (s1 + q2) / E + b1r
    ex2 = (u2 + q3 + 2.0 * b1r * s1) / E + b1r * b1r
    var = ex2 - mean * mean
    s = bn_w[None, :] * lax.rsqrt(var + 1e-5)
    t = bn_b[None, :] - mean * s

    p2 = _tc_p2_head(h1_head, agg1, dis2d, inv2d, b1r, s, t, W2)
    agg2p = _sc_scatter_agg(p2, src3, dst3)

    b2r = b2[None, :]
    red = jnp.full((C, 1), 1.0 / C, f32)
    w2m = jnp.dot(W2, red)
    b2m = jnp.mean(b2).reshape(1, 1)
    tf = t + b1r * s
    y_tail = _tc_tail(h1, s, tf, b2r, W2, w2m, red, b2m,
                      ln_w[None, :], ln_b[None, :], Wl)
    y_head = _tc_head(h1_head, inv2d, agg1, agg2p, dis2d, s, t,
                      b1r, b2r, W2, ln_w[None, :], ln_b[None, :],
                      Wl.reshape(1, C))
    y = jnp.concatenate([y_head[:, 0], y_tail[:, 0]])
    return y + bl[0]


# speedup vs baseline: 1.3611x; 1.1073x over previous
import functools

import jax
import jax.numpy as jnp
from jax import lax
from jax.experimental import pallas as pl
from jax.experimental.pallas import tpu as pltpu
import jax.experimental.pallas.tpu_sc as plsc

N = 10000
E = 320000
C = 128
NC = 2
NS = 16
NW = NC * NS
ECH = E // NC
EW = E // NW
B = 80
NB = EW // B
NPAD = 10240
STR = NPAD // NS
T = 2000
NBLK = N // T

_mesh = plsc.VectorSubcoreMesh(core_axis_name="c", subcore_axis_name="s")


@functools.partial(
    pl.kernel,
    out_type=(
        jax.ShapeDtypeStruct((E, C), jnp.float32),
        jax.ShapeDtypeStruct((NW, C), jnp.float32),
        jax.ShapeDtypeStruct((NW, C), jnp.float32),
        jax.ShapeDtypeStruct((NC, NPAD), jnp.float32),
    ),
    mesh=_mesh,
    scratch_types=[
        pltpu.VMEM((NB, B), jnp.int32),
        pltpu.VMEM((NB, B), jnp.int32),
        pltpu.VMEM((B, C), jnp.float32),
        pltpu.VMEM((B, C), jnp.float32),
        pltpu.VMEM((B, C), jnp.float32),
        pltpu.VMEM((B, C), jnp.float32),
        pltpu.VMEM((B, C), jnp.float32),
        pltpu.VMEM((B, C), jnp.float32),
        pltpu.VMEM((B,), jnp.float32),
        pltpu.VMEM((STR,), jnp.float32),
        pltpu.VMEM((C,), jnp.float32),
        pltpu.VMEM((C,), jnp.float32),
        pltpu.VMEM_SHARED((NPAD,), jnp.float32),
        pltpu.SemaphoreType.DMA,
        pltpu.SemaphoreType.DMA,
        pltpu.SemaphoreType.DMA,
        pltpu.SemaphoreType.DMA,
        pltpu.SemaphoreType.DMA,
        pltpu.SemaphoreType.DMA,
    ],
)
def _sc_build_edges(xa_hbm, xb_hbm, src_hbm, dst_hbm,
                    h1_hbm, s1_hbm, s2_hbm, hist_hbm,
                    srcall, dstall,
                    bufa0, bufb0, out0, bufa1, bufb1, out1,
                    ones_v, zstr_v, acc1_v, acc2_v, hist_sh,
                    ga0, gb0, ga1, gb1, wb0, wb1):
    cid = lax.axis_index("c")
    sid = lax.axis_index("s")
    wid = cid * NS + sid
    cbase = cid * ECH + sid * EW

    for i in range(B // 16):
        ones_v[pl.ds(i * 16, 16)] = jnp.full((16,), 1.0, jnp.float32)

    def _zr(i, _):
        zstr_v[pl.ds(i * 16, 16)] = jnp.zeros((16,), jnp.float32)
        return 0

    lax.fori_loop(0, STR // 16, _zr, 0)
    pltpu.sync_copy(zstr_v, hist_sh.at[pl.ds(sid * STR, STR)])
    pltpu.sync_copy(src_hbm.at[wid], srcall)
    pltpu.sync_copy(dst_hbm.at[wid], dstall)
    plsc.subcore_barrier()

    def _compute(ba, bb, out, accs):
        def _row(i, accs):
            new = list(accs)
            for rr in range(2):
                r = 2 * i + rr
                for c in range(C // 16):
                    sl = pl.ds(c * 16, 16)
                    h = ba[r, sl] + bb[r, sl]
                    out[r, sl] = h
                    new[c] = new[c] + h
                    new[c + 8] = new[c + 8] + h * h
            return tuple(new)

        return lax.fori_loop(0, B // 2, _row, accs)

    def _fetch(b, ba, bb, sa, sb):
        pltpu.async_copy(xa_hbm.at[srcall.at[b]], ba, sa)
        pltpu.async_copy(xb_hbm.at[dstall.at[b]], bb, sb)

    def _gwait(ba, bb, sa, sb):
        pltpu.make_async_copy(xa_hbm.at[srcall.at[0]], ba, sa).wait()
        pltpu.make_async_copy(xb_hbm.at[dstall.at[0]], bb, sb).wait()

    _fetch(0, bufa0, bufb0, ga0, gb0)
    zv = jnp.zeros((16,), jnp.float32)

    def _pair(j, accs):
        _fetch(2 * j + 1, bufa1, bufb1, ga1, gb1)
        _gwait(bufa0, bufb0, ga0, gb0)
        pltpu.sync_copy(ones_v, hist_sh.at[dstall.at[2 * j]], add=True)

        @pl.when(j > 0)
        def _():
            pltpu.make_async_copy(out0, h1_hbm.at[pl.ds(0, B)], wb0).wait()

        accs = _compute(bufa0, bufb0, out0, accs)
        pltpu.async_copy(out0, h1_hbm.at[pl.ds(cbase + 2 * j * B, B)], wb0)
        _fetch(2 * j + 2, bufa0, bufb0, ga0, gb0)
        _gwait(bufa1, bufb1, ga1, gb1)
        pltpu.sync_copy(ones_v, hist_sh.at[dstall.at[2 * j + 1]], add=True)

        @pl.when(j > 0)
        def _():
            pltpu.make_async_copy(out1, h1_hbm.at[pl.ds(0, B)], wb1).wait()

        accs = _compute(bufa1, bufb1, out1, accs)
        pltpu.async_copy(out1, h1_hbm.at[pl.ds(cbase + (2 * j + 1) * B, B)],
                         wb1)
        return accs

    accs = lax.fori_loop(0, (NB - 1) // 2, _pair, (zv,) * 16)

    _gwait(bufa0, bufb0, ga0, gb0)
    pltpu.sync_copy(ones_v, hist_sh.at[dstall.at[NB - 1]], add=True)
    pltpu.make_async_copy(out0, h1_hbm.at[pl.ds(0, B)], wb0).wait()
    accs = _compute(bufa0, bufb0, out0, accs)
    pltpu.async_copy(out0, h1_hbm.at[pl.ds(cbase + (NB - 1) * B, B)], wb0)
    pltpu.make_async_copy(out0, h1_hbm.at[pl.ds(0, B)], wb0).wait()
    pltpu.make_async_copy(out1, h1_hbm.at[pl.ds(0, B)], wb1).wait()

    for c in range(C // 16):
        acc1_v[pl.ds(c * 16, 16)] = accs[c]
        acc2_v[pl.ds(c * 16, 16)] = accs[c + 8]
    pltpu.sync_copy(acc1_v, s1_hbm.at[wid])
    pltpu.sync_copy(acc2_v, s2_hbm.at[wid])
    plsc.subcore_barrier()
    pltpu.sync_copy(hist_sh.at[pl.ds(sid * STR, STR)],
                    hist_hbm.at[cid, pl.ds(sid * STR, STR)])


@functools.partial(
    pl.kernel,
    out_type=jax.ShapeDtypeStruct((NC, NPAD, C), jnp.float32),
    mesh=_mesh,
    scratch_types=(
        [pltpu.VMEM((1, B), jnp.int32) for _ in range(8)]
        + [pltpu.VMEM((B, C), jnp.float32) for _ in range(2)]
        + [pltpu.VMEM((128, C), jnp.float32),
           pltpu.VMEM_SHARED((NPAD, C), jnp.float32)]
        + [pltpu.SemaphoreType.DMA for _ in range(8)]
    ),
)
def _sc_scatter_agg(p_hbm, src_hbm, dst_hbm, out_hbm, *refs):
    idxs = refs[0:4]
    idxd = refs[4:8]
    rows = refs[8:10]
    zero_v = refs[10]
    agg_sh = refs[11]
    isem = refs[12:16]
    gsem = refs[16:18]
    ssem = refs[18:20]
    cid = lax.axis_index("c")
    sid = lax.axis_index("s")
    wid = cid * NS + sid

    def _zr(r, _):
        for c in range(C // 16):
            zero_v[r, pl.ds(c * 16, 16)] = jnp.zeros((16,), jnp.float32)
        return 0

    lax.fori_loop(0, 128, _zr, 0)
    for k in range(STR // 128):
        pltpu.sync_copy(zero_v, agg_sh.at[pl.ds(sid * STR + k * 128, 128)])
    plsc.subcore_barrier()

    def _fire_idx(b, k):
        pltpu.async_copy(src_hbm.at[wid, pl.ds(b, 1)], idxs[k], isem[k])
        pltpu.async_copy(dst_hbm.at[wid, pl.ds(b, 1)], idxd[k], isem[k])

    def _fire_gather(k, p):
        pltpu.make_async_copy(src_hbm.at[wid, pl.ds(0, 1)], idxs[k],
                              isem[k]).wait()
        pltpu.make_async_copy(dst_hbm.at[wid, pl.ds(0, 1)], idxd[k],
                              isem[k]).wait()
        pltpu.async_copy(p_hbm.at[idxs[k].at[0]], rows[p], gsem[p])

    def _gwait(k, p):
        pltpu.make_async_copy(p_hbm.at[idxs[k].at[0]], rows[p],
                              gsem[p]).wait()

    def _step(b, k, p, nk, refill, prefetch):
        _gwait(k, p)
        pltpu.async_copy(rows[p], agg_sh.at[idxd[k].at[0]], ssem[p],
                         add=True)
        pltpu.make_async_copy(rows[p], agg_sh.at[idxd[k].at[0]],
                              ssem[p]).wait()
        if prefetch:
            _fire_idx(b + 4, k)
        if refill:
            _fire_gather(nk, p)

    for k in range(4):
        _fire_idx(k, k)
    _fire_gather(0, 0)
    _fire_gather(1, 1)

    def _quad(q, _):
        b0 = 4 * q
        _step(b0, 0, 0, 2, True, True)
        _step(b0 + 1, 1, 1, 3, True, True)
        _step(b0 + 2, 2, 0, 0, True, True)
        _step(b0 + 3, 3, 1, 1, True, True)
        return 0

    lax.fori_loop(0, (NB - 1) // 4 - 1, _quad, 0)
    b0 = NB - 5
    _step(b0, 0, 0, 2, True, True)
    _step(b0 + 1, 1, 1, 3, True, False)
    _step(b0 + 2, 2, 0, 0, True, False)
    _step(b0 + 3, 3, 1, 1, False, False)
    _step(b0 + 4, 0, 0, 0, False, False)

    plsc.subcore_barrier()
    for k in range(STR // 128):
        pltpu.sync_copy(agg_sh.at[pl.ds(sid * STR + k * 128, 128)],
                        out_hbm.at[cid, pl.ds(sid * STR + k * 128, 128)])


def _tc_xab(x, w1r):
    def body(x_ref, w_ref, o_ref):
        o_ref[...] = jnp.dot(x_ref[...], w_ref[...],
                             preferred_element_type=jnp.float32)

    return pl.pallas_call(
        body,
        out_shape=jax.ShapeDtypeStruct((N, 2 * C), jnp.float32),
    )(x, w1r)


def _tc_scale_head(hs, sdeg2d):
    def body(h_ref, s_ref, o_ref):
        o_ref[...] = h_ref[...] * s_ref[...]

    return pl.pallas_call(
        body,
        grid=(1,),
        in_specs=[
            pl.BlockSpec((N, C), lambda i: (0, 0)),
            pl.BlockSpec((N, 1), lambda i: (0, 0)),
        ],
        out_specs=pl.BlockSpec((N, C), lambda i: (0, 0)),
        out_shape=jax.ShapeDtypeStruct((N, C), jnp.float32),
    )(hs, sdeg2d)


def _tc_combine_agg1(aggp, h1, dis2d, inv2d, b1):
    def body(ap_ref, h_ref, d_ref, iv_ref, b_ref,
             agg_ref, q1_ref, q2_ref, q3_ref):
        a = (ap_ref[0, :N, :] + ap_ref[1, :N, :]) * d_ref[...]
        agg_ref[...] = a
        h = h_ref[...]
        iv = iv_ref[...]
        hi = h * iv
        g = hi + b_ref[...]
        q1_ref[...] = jnp.sum(hi - h, axis=0, keepdims=True)
        q2_ref[...] = jnp.sum(a, axis=0, keepdims=True)
        q3_ref[...] = jnp.sum(hi * hi - h * h + (2.0 * g + a) * a,
                              axis=0, keepdims=True)

    return pl.pallas_call(
        body,
        grid=(1,),
        in_specs=[
            pl.BlockSpec((NC, NPAD, C), lambda i: (0, 0, 0)),
            pl.BlockSpec((N, C), lambda i: (0, 0)),
            pl.BlockSpec((N, 1), lambda i: (0, 0)),
            pl.BlockSpec((N, 1), lambda i: (0, 0)),
            pl.BlockSpec((1, C), lambda i: (0, 0)),
        ],
        out_specs=[
            pl.BlockSpec((N, C), lambda i: (0, 0)),
            pl.BlockSpec((1, C), lambda i: (0, 0)),
            pl.BlockSpec((1, C), lambda i: (0, 0)),
            pl.BlockSpec((1, C), lambda i: (0, 0)),
        ],
        out_shape=[
            jax.ShapeDtypeStruct((N, C), jnp.float32),
            jax.ShapeDtypeStruct((1, C), jnp.float32),
            jax.ShapeDtypeStruct((1, C), jnp.float32),
            jax.ShapeDtypeStruct((1, C), jnp.float32),
        ],
    )(aggp, h1, dis2d, inv2d, b1)


def _tc_p2_head(h1, agg1, dis2d, inv2d, b1, s, t, w2):
    def body(h_ref, a_ref, d_ref, iv_ref, b_ref, s_ref, t_ref, w_ref, o_ref):
        o1 = h_ref[...] * iv_ref[...] + b_ref[...] + a_ref[...]
        z = jnp.maximum(o1 * s_ref[...] + t_ref[...], 0.0)
        o_ref[...] = jnp.dot(z * d_ref[...], w_ref[...],
                             preferred_element_type=jnp.float32)

    return pl.pallas_call(
        body,
        grid=(1,),
        in_specs=[
            pl.BlockSpec((N, C), lambda i: (0, 0)),
            pl.BlockSpec((N, C), lambda i: (0, 0)),
            pl.BlockSpec((N, 1), lambda i: (0, 0)),
            pl.BlockSpec((N, 1), lambda i: (0, 0)),
            pl.BlockSpec((1, C), lambda i: (0, 0)),
            pl.BlockSpec((1, C), lambda i: (0, 0)),
            pl.BlockSpec((1, C), lambda i: (0, 0)),
            pl.BlockSpec((C, C), lambda i: (0, 0)),
        ],
        out_specs=pl.BlockSpec((N, C), lambda i: (0, 0)),
        out_shape=jax.ShapeDtypeStruct((N, C), jnp.float32),
    )(h1, agg1, dis2d, inv2d, b1, s, t, w2)


def _tc_head(h1h, inv2d, agg1, agg2p, dis2d, s, t, b1, b2, w2, lnw, lnb, wlr):
    def body(h_ref, i_ref, a1_ref, a2_ref, d_ref, s_ref, t_ref, b1_ref,
             b2_ref, w2_ref, lw_ref, lb_ref, wl_ref, o_ref):
        o1 = h_ref[...] * i_ref[...] + b1_ref[...] + a1_ref[...]
        z = jnp.maximum(o1 * s_ref[...] + t_ref[...], 0.0)
        h2s = jnp.dot(z * i_ref[...], w2_ref[...],
                      preferred_element_type=jnp.float32)
        a2 = (a2_ref[0] + a2_ref[1]) * d_ref[...]
        o2 = h2s + b2_ref[...] + a2
        mu = jnp.mean(o2, axis=-1, keepdims=True)
        var = jnp.mean(o2 * o2, axis=-1, keepdims=True) - mu * mu
        zz = (o2 - mu) * lax.rsqrt(var + 1e-5) * lw_ref[...] + lb_ref[...]
        zz = jnp.maximum(zz, 0.0)
        o_ref[...] = jnp.sum(zz * wl_ref[...], axis=-1, keepdims=True)

    return pl.pallas_call(
        body,
        grid=(NBLK,),
        in_specs=[
            pl.BlockSpec((T, C), lambda i: (i, 0)),
            pl.BlockSpec((T, 1), lambda i: (i, 0)),
            pl.BlockSpec((T, C), lambda i: (i, 0)),
            pl.BlockSpec((NC, T, C), lambda i: (0, i, 0)),
            pl.BlockSpec((T, 1), lambda i: (i, 0)),
            pl.BlockSpec((1, C), lambda i: (0, 0)),
            pl.BlockSpec((1, C), lambda i: (0, 0)),
            pl.BlockSpec((1, C), lambda i: (0, 0)),
            pl.BlockSpec((1, C), lambda i: (0, 0)),
            pl.BlockSpec((C, C), lambda i: (0, 0)),
            pl.BlockSpec((1, C), lambda i: (0, 0)),
            pl.BlockSpec((1, C), lambda i: (0, 0)),
            pl.BlockSpec((1, C), lambda i: (0, 0)),
        ],
        out_specs=pl.BlockSpec((T, 1), lambda i: (i, 0)),
        out_shape=jax.ShapeDtypeStruct((N, 1), jnp.float32),
    )(h1h, inv2d, agg1, agg2p, dis2d, s, t, b1, b2, w2, lnw, lnb, wlr)


def _tc_tail(h1, sf, tf, b2, w2, w2m, red, b2m, lnw, lnb, wlr):
    def body(h_ref, s_ref, t_ref, b2_ref, w2_ref, wm_ref, rd_ref, bm_ref,
             lw_ref, lb_ref, wl_ref, o_ref):
        z = jnp.maximum(h_ref[...] * s_ref[...] + t_ref[...], 0.0)
        o2 = jnp.dot(z, w2_ref[...], preferred_element_type=jnp.float32)
        o2 = o2 + b2_ref[...]
        mu = jnp.dot(z, wm_ref[...],
                     preferred_element_type=jnp.float32) + bm_ref[...]
        msq = jnp.dot(o2 * o2, rd_ref[...], preferred_element_type=jnp.float32)
        var = msq - mu * mu
        zz = (o2 - mu) * lax.rsqrt(var + 1e-5) * lw_ref[...] + lb_ref[...]
        zz = jnp.maximum(zz, 0.0)
        o_ref[...] = jnp.dot(zz, wl_ref[...], preferred_element_type=jnp.float32)

    return pl.pallas_call(
        body,
        grid=((E - N) // T,),
        in_specs=[
            pl.BlockSpec((T, C), lambda i: (i + NBLK, 0)),
            pl.BlockSpec((1, C), lambda i: (0, 0)),
            pl.BlockSpec((1, C), lambda i: (0, 0)),
            pl.BlockSpec((1, C), lambda i: (0, 0)),
            pl.BlockSpec((C, C), lambda i: (0, 0)),
            pl.BlockSpec((C, 1), lambda i: (0, 0)),
            pl.BlockSpec((C, 1), lambda i: (0, 0)),
            pl.BlockSpec((1, 1), lambda i: (0, 0)),
            pl.BlockSpec((1, C), lambda i: (0, 0)),
            pl.BlockSpec((1, C), lambda i: (0, 0)),
            pl.BlockSpec((C, 1), lambda i: (0, 0)),
        ],
        out_specs=pl.BlockSpec((T, 1), lambda i: (i, 0)),
        out_shape=jax.ShapeDtypeStruct((E - N, 1), jnp.float32),
    )(h1, sf, tf, b2, w2, w2m, red, b2m, lnw, lnb, wlr)


def kernel(x, edge_index, W1, b1, bn_w, bn_b, W2, b2, ln_w, ln_b, Wl, bl):
    f32 = jnp.float32
    src = edge_index[0]
    dst = edge_index[1]

    w1r = jnp.concatenate([W1[:C], W1[C:]], axis=1)
    xab = _tc_xab(x, w1r)
    xa = xab[:, :C]
    xb = xab[:, C:]

    src3 = src.reshape(NW, NB, B)
    dst3 = dst.reshape(NW, NB, B)
    h1, s1p, s2p, hist = _sc_build_edges(xa, xb, src3, dst3)
    u1 = jnp.sum(s1p, axis=0, keepdims=True)
    u2 = jnp.sum(s2p, axis=0, keepdims=True)
    h1_head = jax.lax.slice(h1, (0, 0), (N, C))

    deg = hist[0, :N] + hist[1, :N] + 1.0
    dis = lax.rsqrt(deg)
    inv = 1.0 / deg

    dis2d = dis[:, None]
    inv2d = inv[:, None]
    p1 = _tc_scale_head(h1_head, dis2d)
    agg1p = _sc_scatter_agg(p1, src3, dst3)
    b1r = b1[None, :]
    agg1, q1, q2, q3 = _tc_combine_agg1(agg1p, h1_head, dis2d, inv2d, b1r)

    s1 = u1 + q1
    mean = (s1 + q2) / E + b1r
    ex2 = (u2 + q3 + 2.0 * b1r * s1) / E + b1r * b1r
    var = ex2 - mean * mean
    s = bn_w[None, :] * lax.rsqrt(var + 1e-5)
    t = bn_b[None, :] - mean * s

    p2 = _tc_p2_head(h1_head, agg1, dis2d, inv2d, b1r, s, t, W2)
    agg2p = _sc_scatter_agg(p2, src3, dst3)

    b2r = b2[None, :]
    red = jnp.full((C, 1), 1.0 / C, f32)
    w2m = jnp.dot(W2, red)
    b2m = jnp.mean(b2).reshape(1, 1)
    tf = t + b1r * s
    y_tail = _tc_tail(h1, s, tf, b2r, W2, w2m, red, b2m,
                      ln_w[None, :], ln_b[None, :], Wl)
    y_head = _tc_head(h1_head, inv2d, agg1, agg2p, dis2d, s, t,
                      b1r, b2r, W2, ln_w[None, :], ln_b[None, :],
                      Wl.reshape(1, C))
    y = jnp.concatenate([y_head[:, 0], y_tail[:, 0]])
    return y + bl[0]


# speedup vs baseline: 1.3622x; 1.0008x over previous
import functools

import jax
import jax.numpy as jnp
from jax import lax
from jax.experimental import pallas as pl
from jax.experimental.pallas import tpu as pltpu
import jax.experimental.pallas.tpu_sc as plsc

N = 10000
E = 320000
C = 128
NC = 2
NS = 16
NW = NC * NS
ECH = E // NC
EW = E // NW
B = 80
NB = EW // B
NPAD = 10240
STR = NPAD // NS
T = 2000
NBLK = N // T

_mesh = plsc.VectorSubcoreMesh(core_axis_name="c", subcore_axis_name="s")


@functools.partial(
    pl.kernel,
    out_type=(
        jax.ShapeDtypeStruct((E, C), jnp.float32),
        jax.ShapeDtypeStruct((NW, C), jnp.float32),
        jax.ShapeDtypeStruct((NW, C), jnp.float32),
        jax.ShapeDtypeStruct((NC, NPAD), jnp.float32),
    ),
    mesh=_mesh,
    scratch_types=[
        pltpu.VMEM((NB, B), jnp.int32),
        pltpu.VMEM((NB, B), jnp.int32),
        pltpu.VMEM((B, C), jnp.float32),
        pltpu.VMEM((B, C), jnp.float32),
        pltpu.VMEM((B, C), jnp.float32),
        pltpu.VMEM((B, C), jnp.float32),
        pltpu.VMEM((B, C), jnp.float32),
        pltpu.VMEM((B, C), jnp.float32),
        pltpu.VMEM((B,), jnp.float32),
        pltpu.VMEM((STR,), jnp.float32),
        pltpu.VMEM((C,), jnp.float32),
        pltpu.VMEM((C,), jnp.float32),
        pltpu.VMEM_SHARED((NPAD,), jnp.float32),
        pltpu.SemaphoreType.DMA,
        pltpu.SemaphoreType.DMA,
        pltpu.SemaphoreType.DMA,
        pltpu.SemaphoreType.DMA,
        pltpu.SemaphoreType.DMA,
        pltpu.SemaphoreType.DMA,
        pltpu.SemaphoreType.DMA,
    ],
)
def _sc_build_edges(xa_hbm, xb_hbm, src_hbm, dst_hbm,
                    h1_hbm, s1_hbm, s2_hbm, hist_hbm,
                    srcall, dstall,
                    bufa0, bufb0, out0, bufa1, bufb1, out1,
                    ones_v, zstr_v, acc1_v, acc2_v, hist_sh,
                    ga0, gb0, ga1, gb1, wb0, wb1, hsem):
    cid = lax.axis_index("c")
    sid = lax.axis_index("s")
    wid = cid * NS + sid
    cbase = cid * ECH + sid * EW

    for i in range(B // 16):
        ones_v[pl.ds(i * 16, 16)] = jnp.full((16,), 1.0, jnp.float32)

    def _zr(i, _):
        zstr_v[pl.ds(i * 16, 16)] = jnp.zeros((16,), jnp.float32)
        return 0

    lax.fori_loop(0, STR // 16, _zr, 0)
    pltpu.sync_copy(zstr_v, hist_sh.at[pl.ds(sid * STR, STR)])
    pltpu.sync_copy(src_hbm.at[wid], srcall)
    pltpu.sync_copy(dst_hbm.at[wid], dstall)
    plsc.subcore_barrier()

    def _compute(ba, bb, out, accs):
        def _row(i, accs):
            new = list(accs)
            for rr in range(2):
                r = 2 * i + rr
                for c in range(C // 16):
                    sl = pl.ds(c * 16, 16)
                    h = ba[r, sl] + bb[r, sl]
                    out[r, sl] = h
                    new[c] = new[c] + h
                    new[c + 8] = new[c + 8] + h * h
            return tuple(new)

        return lax.fori_loop(0, B // 2, _row, accs)

    def _fetch(b, ba, bb, sa, sb):
        pltpu.async_copy(xa_hbm.at[srcall.at[b]], ba, sa)
        pltpu.async_copy(xb_hbm.at[dstall.at[b]], bb, sb)

    def _gwait(ba, bb, sa, sb):
        pltpu.make_async_copy(xa_hbm.at[srcall.at[0]], ba, sa).wait()
        pltpu.make_async_copy(xb_hbm.at[dstall.at[0]], bb, sb).wait()

    _fetch(0, bufa0, bufb0, ga0, gb0)
    zv = jnp.zeros((16,), jnp.float32)

    def _pair(j, accs):
        _fetch(2 * j + 1, bufa1, bufb1, ga1, gb1)
        _gwait(bufa0, bufb0, ga0, gb0)
        pltpu.async_copy(ones_v, hist_sh.at[dstall.at[2 * j]], hsem,
                         add=True)

        @pl.when(j > 0)
        def _():
            pltpu.make_async_copy(out0, h1_hbm.at[pl.ds(0, B)], wb0).wait()

        accs = _compute(bufa0, bufb0, out0, accs)
        pltpu.async_copy(out0, h1_hbm.at[pl.ds(cbase + 2 * j * B, B)], wb0)
        _fetch(2 * j + 2, bufa0, bufb0, ga0, gb0)
        _gwait(bufa1, bufb1, ga1, gb1)
        pltpu.async_copy(ones_v, hist_sh.at[dstall.at[2 * j + 1]], hsem,
                         add=True)

        @pl.when(j > 0)
        def _():
            pltpu.make_async_copy(out1, h1_hbm.at[pl.ds(0, B)], wb1).wait()

        accs = _compute(bufa1, bufb1, out1, accs)
        pltpu.async_copy(out1, h1_hbm.at[pl.ds(cbase + (2 * j + 1) * B, B)],
                         wb1)
        return accs

    accs = lax.fori_loop(0, (NB - 1) // 2, _pair, (zv,) * 16)

    _gwait(bufa0, bufb0, ga0, gb0)
    pltpu.async_copy(ones_v, hist_sh.at[dstall.at[NB - 1]], hsem, add=True)
    pltpu.make_async_copy(out0, h1_hbm.at[pl.ds(0, B)], wb0).wait()
    accs = _compute(bufa0, bufb0, out0, accs)
    pltpu.async_copy(out0, h1_hbm.at[pl.ds(cbase + (NB - 1) * B, B)], wb0)
    pltpu.make_async_copy(out0, h1_hbm.at[pl.ds(0, B)], wb0).wait()
    pltpu.make_async_copy(out1, h1_hbm.at[pl.ds(0, B)], wb1).wait()

    for c in range(C // 16):
        acc1_v[pl.ds(c * 16, 16)] = accs[c]
        acc2_v[pl.ds(c * 16, 16)] = accs[c + 8]
    pltpu.sync_copy(acc1_v, s1_hbm.at[wid])
    pltpu.sync_copy(acc2_v, s2_hbm.at[wid])

    def _hdrain(i, _):
        pltpu.make_async_copy(ones_v, hist_sh.at[dstall.at[0]],
                              hsem).wait()
        return 0

    lax.fori_loop(0, NB, _hdrain, 0)
    plsc.subcore_barrier()
    pltpu.sync_copy(hist_sh.at[pl.ds(sid * STR, STR)],
                    hist_hbm.at[cid, pl.ds(sid * STR, STR)])


@functools.partial(
    pl.kernel,
    out_type=jax.ShapeDtypeStruct((NC, NPAD, C), jnp.float32),
    mesh=_mesh,
    scratch_types=(
        [pltpu.VMEM((1, B), jnp.int32) for _ in range(8)]
        + [pltpu.VMEM((B, C), jnp.float32) for _ in range(2)]
        + [pltpu.VMEM((128, C), jnp.float32),
           pltpu.VMEM_SHARED((NPAD, C), jnp.float32)]
        + [pltpu.SemaphoreType.DMA for _ in range(8)]
    ),
)
def _sc_scatter_agg(p_hbm, src_hbm, dst_hbm, out_hbm, *refs):
    idxs = refs[0:4]
    idxd = refs[4:8]
    rows = refs[8:10]
    zero_v = refs[10]
    agg_sh = refs[11]
    isem = refs[12:16]
    gsem = refs[16:18]
    ssem = refs[18:20]
    cid = lax.axis_index("c")
    sid = lax.axis_index("s")
    wid = cid * NS + sid

    def _zr(r, _):
        for c in range(C // 16):
            zero_v[r, pl.ds(c * 16, 16)] = jnp.zeros((16,), jnp.float32)
        return 0

    lax.fori_loop(0, 128, _zr, 0)
    for k in range(STR // 128):
        pltpu.sync_copy(zero_v, agg_sh.at[pl.ds(sid * STR + k * 128, 128)])
    plsc.subcore_barrier()

    def _fire_idx(b, k):
        pltpu.async_copy(src_hbm.at[wid, pl.ds(b, 1)], idxs[k], isem[k])
        pltpu.async_copy(dst_hbm.at[wid, pl.ds(b, 1)], idxd[k], isem[k])

    def _fire_gather(k, p):
        pltpu.make_async_copy(src_hbm.at[wid, pl.ds(0, 1)], idxs[k],
                              isem[k]).wait()
        pltpu.make_async_copy(dst_hbm.at[wid, pl.ds(0, 1)], idxd[k],
                              isem[k]).wait()
        pltpu.async_copy(p_hbm.at[idxs[k].at[0]], rows[p], gsem[p])

    def _gwait(k, p):
        pltpu.make_async_copy(p_hbm.at[idxs[k].at[0]], rows[p],
                              gsem[p]).wait()

    def _step(b, k, p, nk, refill, prefetch):
        _gwait(k, p)
        pltpu.async_copy(rows[p], agg_sh.at[idxd[k].at[0]], ssem[p],
                         add=True)
        pltpu.make_async_copy(rows[p], agg_sh.at[idxd[k].at[0]],
                              ssem[p]).wait()
        if prefetch:
            _fire_idx(b + 4, k)
        if refill:
            _fire_gather(nk, p)

    for k in range(4):
        _fire_idx(k, k)
    _fire_gather(0, 0)
    _fire_gather(1, 1)

    def _quad(q, _):
        b0 = 4 * q
        _step(b0, 0, 0, 2, True, True)
        _step(b0 + 1, 1, 1, 3, True, True)
        _step(b0 + 2, 2, 0, 0, True, True)
        _step(b0 + 3, 3, 1, 1, True, True)
        return 0

    lax.fori_loop(0, (NB - 1) // 4 - 1, _quad, 0)
    b0 = NB - 5
    _step(b0, 0, 0, 2, True, True)
    _step(b0 + 1, 1, 1, 3, True, False)
    _step(b0 + 2, 2, 0, 0, True, False)
    _step(b0 + 3, 3, 1, 1, False, False)
    _step(b0 + 4, 0, 0, 0, False, False)

    plsc.subcore_barrier()
    for k in range(STR // 128):
        pltpu.sync_copy(agg_sh.at[pl.ds(sid * STR + k * 128, 128)],
                        out_hbm.at[cid, pl.ds(sid * STR + k * 128, 128)])


def _tc_xab(x, w1r):
    def body(x_ref, w_ref, o_ref):
        o_ref[...] = jnp.dot(x_ref[...], w_ref[...],
                             preferred_element_type=jnp.float32)

    return pl.pallas_call(
        body,
        out_shape=jax.ShapeDtypeStruct((N, 2 * C), jnp.float32),
    )(x, w1r)


def _tc_scale_head(hs, sdeg2d):
    def body(h_ref, s_ref, o_ref):
        o_ref[...] = h_ref[...] * s_ref[...]

    return pl.pallas_call(
        body,
        grid=(1,),
        in_specs=[
            pl.BlockSpec((N, C), lambda i: (0, 0)),
            pl.BlockSpec((N, 1), lambda i: (0, 0)),
        ],
        out_specs=pl.BlockSpec((N, C), lambda i: (0, 0)),
        out_shape=jax.ShapeDtypeStruct((N, C), jnp.float32),
    )(hs, sdeg2d)


def _tc_combine_agg1(aggp, h1, dis2d, inv2d, b1):
    def body(ap_ref, h_ref, d_ref, iv_ref, b_ref,
             agg_ref, q1_ref, q2_ref, q3_ref):
        a = (ap_ref[0, :N, :] + ap_ref[1, :N, :]) * d_ref[...]
        agg_ref[...] = a
        h = h_ref[...]
        iv = iv_ref[...]
        hi = h * iv
        g = hi + b_ref[...]
        q1_ref[...] = jnp.sum(hi - h, axis=0, keepdims=True)
        q2_ref[...] = jnp.sum(a, axis=0, keepdims=True)
        q3_ref[...] = jnp.sum(hi * hi - h * h + (2.0 * g + a) * a,
                              axis=0, keepdims=True)

    return pl.pallas_call(
        body,
        grid=(1,),
        in_specs=[
            pl.BlockSpec((NC, NPAD, C), lambda i: (0, 0, 0)),
            pl.BlockSpec((N, C), lambda i: (0, 0)),
            pl.BlockSpec((N, 1), lambda i: (0, 0)),
            pl.BlockSpec((N, 1), lambda i: (0, 0)),
            pl.BlockSpec((1, C), lambda i: (0, 0)),
        ],
        out_specs=[
            pl.BlockSpec((N, C), lambda i: (0, 0)),
            pl.BlockSpec((1, C), lambda i: (0, 0)),
            pl.BlockSpec((1, C), lambda i: (0, 0)),
            pl.BlockSpec((1, C), lambda i: (0, 0)),
        ],
        out_shape=[
            jax.ShapeDtypeStruct((N, C), jnp.float32),
            jax.ShapeDtypeStruct((1, C), jnp.float32),
            jax.ShapeDtypeStruct((1, C), jnp.float32),
            jax.ShapeDtypeStruct((1, C), jnp.float32),
        ],
    )(aggp, h1, dis2d, inv2d, b1)


def _tc_p2_head(h1, agg1, dis2d, inv2d, b1, s, t, w2):
    def body(h_ref, a_ref, d_ref, iv_ref, b_ref, s_ref, t_ref, w_ref, o_ref):
        o1 = h_ref[...] * iv_ref[...] + b_ref[...] + a_ref[...]
        z = jnp.maximum(o1 * s_ref[...] + t_ref[...], 0.0)
        o_ref[...] = jnp.dot(z * d_ref[...], w_ref[...],
                             preferred_element_type=jnp.float32)

    return pl.pallas_call(
        body,
        grid=(1,),
        in_specs=[
            pl.BlockSpec((N, C), lambda i: (0, 0)),
            pl.BlockSpec((N, C), lambda i: (0, 0)),
            pl.BlockSpec((N, 1), lambda i: (0, 0)),
            pl.BlockSpec((N, 1), lambda i: (0, 0)),
            pl.BlockSpec((1, C), lambda i: (0, 0)),
            pl.BlockSpec((1, C), lambda i: (0, 0)),
            pl.BlockSpec((1, C), lambda i: (0, 0)),
            pl.BlockSpec((C, C), lambda i: (0, 0)),
        ],
        out_specs=pl.BlockSpec((N, C), lambda i: (0, 0)),
        out_shape=jax.ShapeDtypeStruct((N, C), jnp.float32),
    )(h1, agg1, dis2d, inv2d, b1, s, t, w2)


def _tc_head(h1h, inv2d, agg1, agg2p, dis2d, s, t, b1, b2, w2, lnw, lnb, wlr):
    def body(h_ref, i_ref, a1_ref, a2_ref, d_ref, s_ref, t_ref, b1_ref,
             b2_ref, w2_ref, lw_ref, lb_ref, wl_ref, o_ref):
        o1 = h_ref[...] * i_ref[...] + b1_ref[...] + a1_ref[...]
        z = jnp.maximum(o1 * s_ref[...] + t_ref[...], 0.0)
        h2s = jnp.dot(z * i_ref[...], w2_ref[...],
                      preferred_element_type=jnp.float32)
        a2 = (a2_ref[0] + a2_ref[1]) * d_ref[...]
        o2 = h2s + b2_ref[...] + a2
        mu = jnp.mean(o2, axis=-1, keepdims=True)
        var = jnp.mean(o2 * o2, axis=-1, keepdims=True) - mu * mu
        zz = (o2 - mu) * lax.rsqrt(var + 1e-5) * lw_ref[...] + lb_ref[...]
        zz = jnp.maximum(zz, 0.0)
        o_ref[...] = jnp.sum(zz * wl_ref[...], axis=-1, keepdims=True)

    return pl.pallas_call(
        body,
        grid=(NBLK,),
        in_specs=[
            pl.BlockSpec((T, C), lambda i: (i, 0)),
            pl.BlockSpec((T, 1), lambda i: (i, 0)),
            pl.BlockSpec((T, C), lambda i: (i, 0)),
            pl.BlockSpec((NC, T, C), lambda i: (0, i, 0)),
            pl.BlockSpec((T, 1), lambda i: (i, 0)),
            pl.BlockSpec((1, C), lambda i: (0, 0)),
            pl.BlockSpec((1, C), lambda i: (0, 0)),
            pl.BlockSpec((1, C), lambda i: (0, 0)),
            pl.BlockSpec((1, C), lambda i: (0, 0)),
            pl.BlockSpec((C, C), lambda i: (0, 0)),
            pl.BlockSpec((1, C), lambda i: (0, 0)),
            pl.BlockSpec((1, C), lambda i: (0, 0)),
            pl.BlockSpec((1, C), lambda i: (0, 0)),
        ],
        out_specs=pl.BlockSpec((T, 1), lambda i: (i, 0)),
        out_shape=jax.ShapeDtypeStruct((N, 1), jnp.float32),
    )(h1h, inv2d, agg1, agg2p, dis2d, s, t, b1, b2, w2, lnw, lnb, wlr)


def _tc_tail(h1, sf, tf, b2, w2, w2m, red, b2m, lnw, lnb, wlr):
    def body(h_ref, s_ref, t_ref, b2_ref, w2_ref, wm_ref, rd_ref, bm_ref,
             lw_ref, lb_ref, wl_ref, o_ref):
        z = jnp.maximum(h_ref[...] * s_ref[...] + t_ref[...], 0.0)
        o2 = jnp.dot(z, w2_ref[...], preferred_element_type=jnp.float32)
        o2 = o2 + b2_ref[...]
        mu = jnp.dot(z, wm_ref[...],
                     preferred_element_type=jnp.float32) + bm_ref[...]
        msq = jnp.dot(o2 * o2, rd_ref[...], preferred_element_type=jnp.float32)
        var = msq - mu * mu
        zz = (o2 - mu) * lax.rsqrt(var + 1e-5) * lw_ref[...] + lb_ref[...]
        zz = jnp.maximum(zz, 0.0)
        o_ref[...] = jnp.dot(zz, wl_ref[...], preferred_element_type=jnp.float32)

    return pl.pallas_call(
        body,
        grid=((E - N) // T,),
        in_specs=[
            pl.BlockSpec((T, C), lambda i: (i + NBLK, 0)),
            pl.BlockSpec((1, C), lambda i: (0, 0)),
            pl.BlockSpec((1, C), lambda i: (0, 0)),
            pl.BlockSpec((1, C), lambda i: (0, 0)),
            pl.BlockSpec((C, C), lambda i: (0, 0)),
            pl.BlockSpec((C, 1), lambda i: (0, 0)),
            pl.BlockSpec((C, 1), lambda i: (0, 0)),
            pl.BlockSpec((1, 1), lambda i: (0, 0)),
            pl.BlockSpec((1, C), lambda i: (0, 0)),
            pl.BlockSpec((1, C), lambda i: (0, 0)),
            pl.BlockSpec((C, 1), lambda i: (0, 0)),
        ],
        out_specs=pl.BlockSpec((T, 1), lambda i: (i, 0)),
        out_shape=jax.ShapeDtypeStruct((E - N, 1), jnp.float32),
    )(h1, sf, tf, b2, w2, w2m, red, b2m, lnw, lnb, wlr)


def kernel(x, edge_index, W1, b1, bn_w, bn_b, W2, b2, ln_w, ln_b, Wl, bl):
    f32 = jnp.float32
    src = edge_index[0]
    dst = edge_index[1]

    w1r = jnp.concatenate([W1[:C], W1[C:]], axis=1)
    xab = _tc_xab(x, w1r)
    xa = xab[:, :C]
    xb = xab[:, C:]

    src3 = src.reshape(NW, NB, B)
    dst3 = dst.reshape(NW, NB, B)
    h1, s1p, s2p, hist = _sc_build_edges(xa, xb, src3, dst3)
    u1 = jnp.sum(s1p, axis=0, keepdims=True)
    u2 = jnp.sum(s2p, axis=0, keepdims=True)
    h1_head = jax.lax.slice(h1, (0, 0), (N, C))

    deg = hist[0, :N] + hist[1, :N] + 1.0
    dis = lax.rsqrt(deg)
    inv = 1.0 / deg

    dis2d = dis[:, None]
    inv2d = inv[:, None]
    p1 = _tc_scale_head(h1_head, dis2d)
    agg1p = _sc_scatter_agg(p1, src3, dst3)
    b1r = b1[None, :]
    agg1, q1, q2, q3 = _tc_combine_agg1(agg1p, h1_head, dis2d, inv2d, b1r)

    s1 = u1 + q1
    mean = (s1 + q2) / E + b1r
    ex2 = (u2 + q3 + 2.0 * b1r * s1) / E + b1r * b1r
    var = ex2 - mean * mean
    s = bn_w[None, :] * lax.rsqrt(var + 1e-5)
    t = bn_b[None, :] - mean * s

    p2 = _tc_p2_head(h1_head, agg1, dis2d, inv2d, b1r, s, t, W2)
    agg2p = _sc_scatter_agg(p2, src3, dst3)

    b2r = b2[None, :]
    red = jnp.full((C, 1), 1.0 / C, f32)
    w2m = jnp.dot(W2, red)
    b2m = jnp.mean(b2).reshape(1, 1)
    tf = t + b1r * s
    y_tail = _tc_tail(h1, s, tf, b2r, W2, w2m, red, b2m,
                      ln_w[None, :], ln_b[None, :], Wl)
    y_head = _tc_head(h1_head, inv2d, agg1, agg2p, dis2d, s, t,
                      b1r, b2r, W2, ln_w[None, :], ln_b[None, :],
                      Wl.reshape(1, C))
    y = jnp.concatenate([y_head[:, 0], y_tail[:, 0]])
    return y + bl[0]
